# Initial kernel scaffold; baseline (speedup 1.0000x reference)
import jax
import jax.numpy as jnp
from jax.experimental import pallas as pl


def kernel(x, spatial_size, location):
    raise NotImplementedError("write your pallas kernel here")


# speedup vs baseline: 2.0199x; 2.0199x over previous
import functools

import jax
import jax.numpy as jnp
from jax import lax
from jax.experimental import pallas as pl
from jax.experimental.pallas import tpu as pltpu
from jax.experimental.pallas import tpu_sc as plsc

_B, _M, _N = 32, 512, 32
_HW = 256
_P = _HW * _HW
_L = 16
_NC = 2


def _scatter_planes(x, loc):
    mesh = plsc.VectorSubcoreMesh(core_axis_name="c", subcore_axis_name="s")

    @functools.partial(
        pl.kernel,
        mesh=mesh,
        out_type=jax.ShapeDtypeStruct((_B, _N, _P), jnp.float32),
        compiler_params=pltpu.CompilerParams(needs_layout_passes=False),
        scratch_types=[
            pltpu.VMEM((_M * 2,), jnp.int32),
            pltpu.VMEM((_M,), jnp.int32),
            pltpu.VMEM((_M * _N,), jnp.float32),
            pltpu.VMEM((_P,), jnp.float32),
        ],
    )
    def k(x_hbm, loc_hbm, out_hbm, loc_v, flat_v, x_v, plane_v):
        b = lax.axis_index("s") * _NC + lax.axis_index("c")

        pltpu.sync_copy(x_hbm.at[b], x_v)
        pltpu.sync_copy(loc_hbm.at[b], loc_v)

        lane = lax.iota(jnp.int32, _L)
        zero_f = jnp.zeros((_L,), jnp.float32)

        def flat_body(g, c):
            m_idx = (g * _L + lane) * 2
            y = plsc.load_gather(loc_v, [m_idx])
            xc = plsc.load_gather(loc_v, [m_idx + 1])
            flat_v[pl.ds(g * _L, _L)] = y * _HW + xc
            return c

        lax.fori_loop(0, _M // _L, flat_body, 0)

        def zero_body(i, c):
            plane_v[pl.ds(i * _L, _L)] = zero_f
            return c

        lax.fori_loop(0, _P // _L, zero_body, 0)

        def plane_body(n, c):
            n_vec = jnp.full((_L,), n, jnp.int32)
            for g in range(_M // _L):
                m_idx = (g * _L + lane) * _N + n_vec
                f = flat_v[pl.ds(g * _L, _L)]
                val = plsc.load_gather(x_v, [m_idx])
                plsc.addupdate_scatter(plane_v, [f], val)
            pltpu.sync_copy(plane_v, out_hbm.at[b, n])
            for g in range(_M // _L):
                f = flat_v[pl.ds(g * _L, _L)]
                plsc.store_scatter(plane_v, [f], zero_f)
            return c

        lax.fori_loop(0, _N, plane_body, 0)

    return k(x, loc)


def kernel(x, spatial_size, location):
    loc = location.astype(jnp.int32).reshape(_B, _M * 2)
    out = _scatter_planes(x.reshape(_B, _M * _N), loc)
    return out.reshape(_B, _N, _HW, _HW)
